# baseline (speedup 1.0000x reference)
import functools

import jax
import jax.numpy as jnp
from jax import lax
from jax.experimental import pallas as pl
from jax.experimental.pallas import tpu as pltpu
from jax.experimental.pallas import tpu_sc as plsc

_BATCH = 16384
_HIST = 50
_DIM = 32
_N = _BATCH * _HIST
_NC = 2
_NS = 16
_NW = _NC * _NS
_BB = _BATCH // _NW
_L = 16


@functools.partial(
    pl.kernel,
    out_type=jax.ShapeDtypeStruct((_HIST, _DIM, _BATCH), jnp.float32),
    mesh=plsc.VectorSubcoreMesh(core_axis_name="c", subcore_axis_name="s"),
    scratch_types=[
        pltpu.VMEM((_BB * _HIST,), jnp.int32),
        pltpu.VMEM((_HIST, _BB), jnp.int32),
        pltpu.VMEM((_BB, _DIM), jnp.float32),
        pltpu.VMEM((_BB, _DIM), jnp.float32),
        pltpu.VMEM((_DIM, _BB + 17), jnp.float32),
        pltpu.VMEM((_DIM, _BB + 17), jnp.float32),
        pltpu.SemaphoreType.DMA,
        pltpu.SemaphoreType.DMA,
        pltpu.SemaphoreType.DMA,
        pltpu.SemaphoreType.DMA,
    ],
    compiler_params=pltpu.CompilerParams(use_tc_tiling_on_sc=False, needs_layout_passes=False),
)
def _gather_kernel(idx_hbm, table_hbm, out_hbm, idx_v, idxT, rows0, rows1,
                   t0, t1, g0, g1, s0, s1):
    wid = lax.axis_index("s") * _NC + lax.axis_index("c")
    b0 = wid * _BB
    rows = (rows0, rows1)
    tbuf = (t0, t1)
    gsem = (g0, g1)
    ssem = (s0, s1)
    iota = lax.iota(jnp.int32, _L)
    cols = tuple(jnp.full((_L,), j, jnp.int32) for j in range(_DIM))

    pltpu.sync_copy(idx_hbm.at[pl.ds(b0 * _HIST, _BB * _HIST)], idx_v)

    def idx_t(h, _):
        @plsc.parallel_loop(0, _BB // _L, unroll=2)
        def _(k):
            src = (k * _L + iota) * _HIST + h
            idxT[h, pl.ds(k * _L, _L)] = plsc.load_gather(idx_v, [src])
        return ()
    lax.fori_loop(0, _HIST, idx_t, ())

    def gather(h, b):
        return pltpu.async_copy(table_hbm.at[idxT.at[h]], rows[b], gsem[b])

    def gather_wait(h, b):
        pltpu.make_async_copy(
            table_hbm.at[idxT.at[h]], rows[b], gsem[b]).wait()

    def store(h, b):
        return pltpu.async_copy(
            tbuf[b].at[:, pl.ds(0, _BB)],
            out_hbm.at[h, :, pl.ds(b0, _BB)], ssem[b])

    def store_wait(h, b):
        pltpu.make_async_copy(
            tbuf[b].at[:, pl.ds(0, _BB)],
            out_hbm.at[h, :, pl.ds(b0, _BB)], ssem[b]).wait()

    gather(0, 0)

    def pair(g, _):
        for b in (0, 1):
            h = 2 * g + b
            gather_wait(h, b)
            @pl.when(h + 1 < _HIST)
            def _():
                gather(h + 1, 1 - b)
            @pl.when(h >= 2)
            def _():
                store_wait(h - 2, b)

            @plsc.parallel_loop(0, _BB, unroll=4)
            def _(r, b=b):
                col = jnp.broadcast_to(r, (_L,)).astype(jnp.int32)
                for u in range(_DIM // _L):
                    v = rows[b][r, pl.ds(u * _L, _L)]
                    plsc.store_scatter(tbuf[b], [iota + u * _L, col], v)

            store(h, b)
        return ()

    lax.fori_loop(0, _HIST // 2, pair, ())
    store_wait(_HIST - 2, 0)
    store_wait(_HIST - 1, 1)


_VOCAB = 1000000
_TCOLS = 31248
_TC_CH = 496
_TC_N = _TCOLS // _TC_CH
_TAIL = _VOCAB - _NW * _TCOLS


@functools.partial(
    pl.kernel,
    out_type=jax.ShapeDtypeStruct((_VOCAB, _DIM), jnp.float32),
    mesh=plsc.VectorSubcoreMesh(core_axis_name="c", subcore_axis_name="s"),
    scratch_types=[
        pltpu.VMEM((_DIM, _TC_CH), jnp.float32),
        pltpu.VMEM((_DIM, _TC_CH), jnp.float32),
        pltpu.VMEM((_TC_CH, _DIM + 1), jnp.float32),
        pltpu.VMEM((_TC_CH, _DIM + 1), jnp.float32),
        pltpu.SemaphoreType.DMA,
        pltpu.SemaphoreType.DMA,
        pltpu.SemaphoreType.DMA,
        pltpu.SemaphoreType.DMA,
    ],
    compiler_params=pltpu.CompilerParams(
        use_tc_tiling_on_sc=False, needs_layout_passes=False),
)
def _transpose_kernel(tt_hbm, out_hbm, in0, in1, o0, o1, g0, g1, s0, s1):
    wid = lax.axis_index("s") * _NC + lax.axis_index("c")
    c0 = wid * _TCOLS
    ibuf = (in0, in1)
    obuf = (o0, o1)
    gsem = (g0, g1)
    ssem = (s0, s1)
    iota = lax.iota(jnp.int32, _L)

    def load(c, b):
        return pltpu.async_copy(
            tt_hbm.at[:, pl.ds(c0 + c * _TC_CH, _TC_CH)], ibuf[b], gsem[b])

    def load_wait(c, b):
        pltpu.make_async_copy(
            tt_hbm.at[:, pl.ds(c0 + c * _TC_CH, _TC_CH)], ibuf[b],
            gsem[b]).wait()

    def store(c, b):
        return pltpu.async_copy(
            obuf[b].at[:, pl.ds(0, _DIM)],
            out_hbm.at[pl.ds(c0 + c * _TC_CH, _TC_CH)], ssem[b])

    def store_wait(c, b):
        pltpu.make_async_copy(
            obuf[b].at[:, pl.ds(0, _DIM)],
            out_hbm.at[pl.ds(c0 + c * _TC_CH, _TC_CH)], ssem[b]).wait()

    def transpose_chunk(b):
        for j in range(_DIM):
            colv = jnp.full((_L,), j, jnp.int32)
            @plsc.parallel_loop(0, _TC_CH // _L, unroll=4)
            def _(k, b=b, j=j, colv=colv):
                v = ibuf[b][j, pl.ds(k * _L, _L)]
                plsc.store_scatter(obuf[b], [k * _L + iota, colv], v)

    load(0, 0)

    def pair(g, _):
        for b in (0, 1):
            c = 2 * g + b
            load_wait(c, b)
            @pl.when(c + 1 < _TC_N)
            def _():
                load(c + 1, 1 - b)
            @pl.when(c >= 2)
            def _():
                store_wait(c - 2, b)
            transpose_chunk(b)
            store(c, b)
        return ()

    lax.fori_loop(0, (_TC_N - 1) // 2, pair, ())
    c_last = _TC_N - 1
    load_wait(c_last, 0)
    store_wait(c_last - 2, 0)
    transpose_chunk(0)
    store(c_last, 0)
    store_wait(c_last - 1, 1)
    store_wait(c_last, 0)

    @pl.when(wid == _NW - 1)
    def _():
        t0 = _NW * _TCOLS
        pltpu.sync_copy(tt_hbm.at[:, pl.ds(t0, _TAIL)],
                        ibuf[1].at[:, pl.ds(0, _TAIL)])
        for j in range(_DIM):
            colv = jnp.full((_L,), j, jnp.int32)
            for k in range(_TAIL // _L):
                v = ibuf[1][j, pl.ds(k * _L, _L)]
                plsc.store_scatter(obuf[1], [k * _L + iota, colv], v)
        pltpu.sync_copy(obuf[1].at[pl.ds(0, _TAIL), pl.ds(0, _DIM)],
                        out_hbm.at[pl.ds(t0, _TAIL)])


def kernel(batch, table):
    idx = batch.reshape(_N).astype(jnp.int32)
    table_lin = _transpose_kernel(table.T)
    out = _gather_kernel(idx, table_lin)
    return out.transpose(2, 0, 1)


# speedup vs baseline: 4.3894x; 4.3894x over previous
import functools

import jax
import jax.numpy as jnp
from jax import lax
from jax.experimental import pallas as pl
from jax.experimental.pallas import tpu as pltpu
from jax.experimental.pallas import tpu_sc as plsc

_BATCH = 16384
_HIST = 50
_DIM = 32
_N = _BATCH * _HIST
_NC = 2
_NS = 16
_NW = _NC * _NS
_BB = _BATCH // _NW
_L = 16


@functools.partial(
    pl.kernel,
    out_type=jax.ShapeDtypeStruct((_HIST, _DIM, _BATCH), jnp.float32),
    mesh=plsc.VectorSubcoreMesh(core_axis_name="c", subcore_axis_name="s"),
    scratch_types=[
        pltpu.VMEM((_BB * _HIST,), jnp.int32),
        pltpu.VMEM((_HIST, _BB), jnp.int32),
        pltpu.VMEM((_BB, _DIM), jnp.float32),
        pltpu.VMEM((_BB, _DIM), jnp.float32),
        pltpu.VMEM((_DIM, _BB + 17), jnp.float32),
        pltpu.VMEM((_DIM, _BB + 17), jnp.float32),
        pltpu.SemaphoreType.DMA,
        pltpu.SemaphoreType.DMA,
        pltpu.SemaphoreType.DMA,
        pltpu.SemaphoreType.DMA,
    ],
    compiler_params=pltpu.CompilerParams(use_tc_tiling_on_sc=False, needs_layout_passes=False),
)
def _gather_kernel(idx_hbm, table_hbm, out_hbm, idx_v, idxT, rows0, rows1,
                   t0, t1, g0, g1, s0, s1):
    wid = lax.axis_index("s") * _NC + lax.axis_index("c")
    b0 = wid * _BB
    rows = (rows0, rows1)
    tbuf = (t0, t1)
    gsem = (g0, g1)
    ssem = (s0, s1)
    iota = lax.iota(jnp.int32, _L)
    cols = tuple(jnp.full((_L,), j, jnp.int32) for j in range(_DIM))

    pltpu.sync_copy(idx_hbm.at[pl.ds(b0 * _HIST, _BB * _HIST)], idx_v)

    def idx_t(h, _):
        @plsc.parallel_loop(0, _BB // _L, unroll=2)
        def _(k):
            src = (k * _L + iota) * _HIST + h
            idxT[h, pl.ds(k * _L, _L)] = plsc.load_gather(idx_v, [src])
        return ()
    lax.fori_loop(0, _HIST, idx_t, ())

    def gather(h, b):
        return pltpu.async_copy(table_hbm.at[idxT.at[h]], rows[b], gsem[b])

    def gather_wait(h, b):
        pltpu.make_async_copy(
            table_hbm.at[idxT.at[h]], rows[b], gsem[b]).wait()

    def store(h, b):
        return pltpu.async_copy(
            tbuf[b].at[:, pl.ds(0, _BB)],
            out_hbm.at[h, :, pl.ds(b0, _BB)], ssem[b])

    def store_wait(h, b):
        pltpu.make_async_copy(
            tbuf[b].at[:, pl.ds(0, _BB)],
            out_hbm.at[h, :, pl.ds(b0, _BB)], ssem[b]).wait()

    gather(0, 0)

    def pair(g, _):
        for b in (0, 1):
            h = 2 * g + b
            gather_wait(h, b)
            @pl.when(h + 1 < _HIST)
            def _():
                gather(h + 1, 1 - b)
            @pl.when(h >= 2)
            def _():
                store_wait(h - 2, b)

            @plsc.parallel_loop(0, _BB, unroll=8)
            def _(r, b=b):
                col = jnp.broadcast_to(r, (_L,)).astype(jnp.int32)
                for u in range(_DIM // _L):
                    v = rows[b][r, pl.ds(u * _L, _L)]
                    plsc.store_scatter(tbuf[b], [iota + u * _L, col], v)

            store(h, b)
        return ()

    lax.fori_loop(0, _HIST // 2, pair, ())
    store_wait(_HIST - 2, 0)
    store_wait(_HIST - 1, 1)


def kernel(batch, table):
    idx = batch.reshape(_N).astype(jnp.int32)
    out = _gather_kernel(idx, table)
    return out.transpose(2, 0, 1)
